# Initial kernel scaffold; baseline (speedup 1.0000x reference)
import jax
import jax.numpy as jnp
from jax.experimental import pallas as pl


def kernel(log_probs, alive_scores, k):
    raise NotImplementedError("write your pallas kernel here")


# speedup vs baseline: 2.3345x; 2.3345x over previous
import jax
import jax.numpy as jnp
from jax.experimental import pallas as pl

B = 32
BEAM = 16
V = 32768
K = 16
EOS_INDEX = 2
EOS_THRESHOLD = 1.5
NEG_BIG = -1e20
NEG_SENTINEL = -3e38


def _beam_step_kernel(lp_ref, alive_ref, vals_ref, beam_ref, tok_ref):
    x = lp_ref[0]
    alive = alive_ref[0]

    m = jnp.max(x, axis=-1, keepdims=True)
    s = jnp.sum(jnp.exp(x - m), axis=-1, keepdims=True)
    lse = m + jnp.log(s)

    col = jax.lax.broadcasted_iota(jnp.int32, (BEAM, V), 1)
    row = jax.lax.broadcasted_iota(jnp.int32, (BEAM, V), 0)
    flat = row * V + col

    lp = x - lse
    max_lp = m - lse
    eos_lp = jnp.max(jnp.where(col == EOS_INDEX, lp, jnp.float32(NEG_SENTINEL)),
                     axis=-1, keepdims=True)
    cond = eos_lp > (jnp.float32(EOS_THRESHOLD) * max_lp)
    masked_eos = jnp.where(cond, eos_lp, jnp.float32(NEG_BIG))
    lp = jnp.where(col == EOS_INDEX, masked_eos, lp)

    scores = lp + jnp.transpose(alive)

    oiota = jax.lax.broadcasted_iota(jnp.int32, (1, K), 1)
    vals_acc = jnp.zeros((1, K), jnp.float32)
    beam_acc = jnp.zeros((1, K), jnp.int32)
    tok_acc = jnp.zeros((1, K), jnp.int32)
    for i in range(K):
        g = jnp.max(scores)
        eq = scores == g
        fi = jnp.min(jnp.where(eq, flat, jnp.int32(2 ** 30)))
        vals_acc = jnp.where(oiota == i, g, vals_acc)
        beam_acc = jnp.where(oiota == i, fi // V, beam_acc)
        tok_acc = jnp.where(oiota == i, fi % V, tok_acc)
        scores = jnp.where(flat == fi, jnp.float32(NEG_SENTINEL), scores)

    vals_ref[0] = vals_acc
    beam_ref[0] = beam_acc
    tok_ref[0] = tok_acc


def kernel(log_probs, alive_scores, k):
    del k
    alive3 = alive_scores.reshape(B, 1, BEAM)
    out_shapes = [
        jax.ShapeDtypeStruct((B, 1, K), jnp.float32),
        jax.ShapeDtypeStruct((B, 1, K), jnp.int32),
        jax.ShapeDtypeStruct((B, 1, K), jnp.int32),
    ]
    out_spec = pl.BlockSpec((1, 1, K), lambda b: (b, 0, 0))
    topv, beam_idx, tok_idx = pl.pallas_call(
        _beam_step_kernel,
        grid=(B,),
        in_specs=[
            pl.BlockSpec((1, BEAM, V), lambda b: (b, 0, 0)),
            pl.BlockSpec((1, 1, BEAM), lambda b: (b, 0, 0)),
        ],
        out_specs=[out_spec, out_spec, out_spec],
        out_shape=out_shapes,
    )(log_probs, alive3)
    return (topv.reshape(B, K), beam_idx.reshape(B, K), tok_idx.reshape(B, K))


# speedup vs baseline: 4.6600x; 1.9962x over previous
import jax
import jax.numpy as jnp
from jax.experimental import pallas as pl
from jax.experimental.pallas import tpu as pltpu

B = 32
BEAM = 16
V = 32768
R = 256
L = 128
K = 16
EOS_INDEX = 2
EOS_THRESHOLD = 1.5
NEG_BIG = -1e20
NEG_SENT = -3e38
BIG_I = 2 ** 30


def _beam_step_kernel(lp_ref, alive_ref, vals_ref, beam_ref, tok_ref, xs_ref):
    x = lp_ref[0]
    alive = jnp.transpose(alive_ref[0])

    c0 = jnp.max(x, axis=1)
    m = jnp.max(c0, axis=1, keepdims=True)
    sumexp = jnp.sum(jnp.exp(x - m[:, :, None]), axis=(1, 2), keepdims=False)
    lse = m + jnp.log(sumexp)[:, None]

    lane2 = jax.lax.broadcasted_iota(jnp.int32, (BEAM, L), 1)
    row0 = x[:, 0, :]
    eos_x = jnp.max(jnp.where(lane2 == EOS_INDEX, row0, jnp.float32(NEG_SENT)),
                    axis=1, keepdims=True)
    eos_lp = eos_x - lse
    max_lp = m - lse
    cond = eos_lp > (jnp.float32(EOS_THRESHOLD) * max_lp)
    eos_new = jnp.where(cond, eos_x, jnp.float32(NEG_BIG) + lse - alive)

    riota3 = jax.lax.broadcasted_iota(jnp.int32, (BEAM, R, L), 1)
    liota3 = jax.lax.broadcasted_iota(jnp.int32, (BEAM, R, L), 2)
    biota3 = jax.lax.broadcasted_iota(jnp.int32, (BEAM, R, L), 0)
    xs = jnp.where((riota3 == 0) & (liota3 == EOS_INDEX),
                   eos_new[:, :, None], x)
    xs_ref[...] = xs

    flat3 = biota3 * V + riota3 * L + liota3
    c_raw = jnp.max(xs, axis=1)
    eqc = xs == c_raw[:, None, :]
    f = jnp.min(jnp.where(eqc, flat3, BIG_I), axis=1)
    shift = alive - lse
    c = c_raw + shift

    oiota = jax.lax.broadcasted_iota(jnp.int32, (1, K), 1)
    b16 = jax.lax.broadcasted_iota(jnp.int32, (BEAM, 1), 0)
    l1 = jax.lax.broadcasted_iota(jnp.int32, (1, L), 1)
    r2 = jax.lax.broadcasted_iota(jnp.int32, (1, R, L), 1)
    l2 = jax.lax.broadcasted_iota(jnp.int32, (1, R, L), 2)
    vals_acc = jnp.zeros((1, K), jnp.float32)
    beam_acc = jnp.zeros((1, K), jnp.int32)
    tok_acc = jnp.zeros((1, K), jnp.int32)
    for i in range(K):
        g = jnp.max(c)
        fi = jnp.min(jnp.where(c == g, f, BIG_I))
        bi = fi // V
        v = fi - bi * V
        r = v // L
        lw = v - r * L
        vals_acc = jnp.where(oiota == i, g, vals_acc)
        beam_acc = jnp.where(oiota == i, bi, beam_acc)
        tok_acc = jnp.where(oiota == i, v, tok_acc)
        row = xs_ref[bi, pl.ds(r, 1), :]
        xs_ref[bi, pl.ds(r, 1), :] = jnp.where(l1 == lw,
                                               jnp.float32(NEG_SENT), row)
        xb = xs_ref[pl.ds(bi, 1)]
        crow_raw = jnp.max(xb, axis=1)
        eqr = xb == crow_raw[:, None, :]
        frow = jnp.min(jnp.where(eqr, bi * V + r2 * L + l2, BIG_I), axis=1)
        shift_bi = jnp.max(jnp.where(b16 == bi, shift, jnp.float32(NEG_SENT)))
        c = jnp.where(b16 == bi, crow_raw + shift_bi, c)
        f = jnp.where(b16 == bi, frow, f)

    vals_ref[0] = vals_acc
    beam_ref[0] = beam_acc
    tok_ref[0] = tok_acc


def kernel(log_probs, alive_scores, k):
    del k
    lp4 = log_probs.reshape(B, BEAM, R, L)
    alive3 = alive_scores.reshape(B, 1, BEAM)
    out_shapes = [
        jax.ShapeDtypeStruct((B, 1, K), jnp.float32),
        jax.ShapeDtypeStruct((B, 1, K), jnp.int32),
        jax.ShapeDtypeStruct((B, 1, K), jnp.int32),
    ]
    out_spec = pl.BlockSpec((1, 1, K), lambda b: (b, 0, 0))
    topv, beam_idx, tok_idx = pl.pallas_call(
        _beam_step_kernel,
        grid=(B,),
        in_specs=[
            pl.BlockSpec((1, BEAM, R, L), lambda b: (b, 0, 0, 0)),
            pl.BlockSpec((1, 1, BEAM), lambda b: (b, 0, 0)),
        ],
        out_specs=[out_spec, out_spec, out_spec],
        out_shape=out_shapes,
        scratch_shapes=[pltpu.VMEM((BEAM, R, L), jnp.float32)],
    )(lp4, alive3)
    return (topv.reshape(B, K), beam_idx.reshape(B, K), tok_idx.reshape(B, K))


# speedup vs baseline: 5.3744x; 1.1533x over previous
import jax
import jax.numpy as jnp
from jax.experimental import pallas as pl
from jax.experimental.pallas import tpu as pltpu

B = 32
BEAM = 16
V = 32768
K = 16
EOS_INDEX = 2
EOS_THRESHOLD = 1.5
NEG_BIG = -1e20
NEG_SENT = -3e38
BIG_I = 2 ** 30

BP = 4
SEG = 4
RS = 64
NQ = BP * BEAM * SEG
NB = BP * BEAM


def _beam_step_kernel(lp_ref, alive_ref, vals_ref, beam_ref, tok_ref, xs_ref):
    x = lp_ref[...]
    alive = jnp.max(alive_ref[...].reshape(NB, 128), axis=1, keepdims=True)

    c0 = jnp.max(x, axis=1)
    m_beam = jnp.max(c0.reshape(NB, SEG, 128), axis=(1, 2), keepdims=True)
    m_q = jnp.broadcast_to(m_beam, (NB, SEG, 1)).reshape(NQ, 1)
    sumexp_q = jnp.sum(jnp.exp(x - m_q[:, :, None]), axis=(1, 2))
    sumexp = jnp.sum(sumexp_q.reshape(NB, SEG, 1), axis=1)
    lse = m_beam.reshape(NB, 1) + jnp.log(sumexp)
    shift = alive - lse

    lane2 = jax.lax.broadcasted_iota(jnp.int32, (NQ, 128), 1)
    e_row = jnp.max(jnp.where(lane2 == EOS_INDEX, x[:, 0, :],
                              jnp.float32(NEG_SENT)), axis=1, keepdims=True)
    eos_x = e_row.reshape(NB, SEG, 1)[:, 0, :]
    cond = (eos_x - lse) > (jnp.float32(EOS_THRESHOLD)
                            * (m_beam.reshape(NB, 1) - lse))
    eos_new = jnp.where(cond, eos_x, jnp.float32(NEG_BIG) - shift)
    eos_new_q = jnp.broadcast_to(eos_new[:, None, :],
                                 (NB, SEG, 1)).reshape(NQ, 1)

    qio = jax.lax.broadcasted_iota(jnp.int32, (NQ, RS, 128), 0)
    rio = jax.lax.broadcasted_iota(jnp.int32, (NQ, RS, 128), 1)
    lio = jax.lax.broadcasted_iota(jnp.int32, (NQ, RS, 128), 2)
    patch = ((qio & 3) == 0) & (rio == 0) & (lio == EOS_INDEX)
    xs = jnp.where(patch, eos_new_q[:, :, None], x)
    xs_ref[...] = xs

    c_raw = jnp.max(xs, axis=1)
    flat3 = ((qio >> 2) & (BEAM - 1)) * V + (qio & 3) * (RS * 128) \
        + rio * 128 + lio
    f_tab = jnp.min(jnp.where(xs == c_raw[:, None, :], flat3, BIG_I),
                    axis=1)
    shift_q = jnp.broadcast_to(shift[:, None, :], (NB, SEG, 1)).reshape(NQ, 1)
    c_tab = c_raw + shift_q

    cs = [c_tab[b * 64:(b + 1) * 64] for b in range(BP)]
    fs = [f_tab[b * 64:(b + 1) * 64] for b in range(BP)]

    oiota = jax.lax.broadcasted_iota(jnp.int32, (1, K), 1)
    rows64 = jax.lax.broadcasted_iota(jnp.int32, (64, 1), 0)
    rowsNB = jax.lax.broadcasted_iota(jnp.int32, (NB, 1), 0)
    l1 = jax.lax.broadcasted_iota(jnp.int32, (1, 128), 1)
    r2 = jax.lax.broadcasted_iota(jnp.int32, (1, RS, 128), 1)
    l2 = jax.lax.broadcasted_iota(jnp.int32, (1, RS, 128), 2)
    accs = [[jnp.zeros((1, K), jnp.float32), jnp.zeros((1, K), jnp.int32),
             jnp.zeros((1, K), jnp.int32)] for _ in range(BP)]
    for i in range(K):
        for b in range(BP):
            g = jnp.max(cs[b])
            fi = jnp.min(jnp.where(cs[b] == g, fs[b], BIG_I))
            bi = fi >> 15
            v = fi & (V - 1)
            s = v >> 13
            r = (v >> 7) & (RS - 1)
            lw = v & 127
            q = b * 64 + bi * SEG + s
            accs[b][0] = jnp.where(oiota == i, g, accs[b][0])
            accs[b][1] = jnp.where(oiota == i, bi, accs[b][1])
            accs[b][2] = jnp.where(oiota == i, v, accs[b][2])
            row = xs_ref[q, pl.ds(r, 1), :]
            xs_ref[q, pl.ds(r, 1), :] = jnp.where(
                l1 == lw, jnp.float32(NEG_SENT), row)
            xq = xs_ref[pl.ds(q, 1)]
            crow_raw = jnp.max(xq, axis=1)
            frow = jnp.min(jnp.where(xq == crow_raw[:, None, :],
                                     bi * V + s * (RS * 128) + r2 * 128 + l2,
                                     BIG_I), axis=1)
            shift_b = jnp.max(jnp.where(rowsNB == b * BEAM + bi,
                                        shift, jnp.float32(NEG_SENT)))
            sel = rows64 == bi * SEG + s
            cs[b] = jnp.where(sel, crow_raw + shift_b, cs[b])
            fs[b] = jnp.where(sel, frow, fs[b])

    for b in range(BP):
        vals_ref[b] = accs[b][0]
        beam_ref[b] = accs[b][1]
        tok_ref[b] = accs[b][2]


def kernel(log_probs, alive_scores, k):
    del k
    lp3 = log_probs.reshape(B * BEAM * SEG, RS, 128)
    alive3 = jnp.broadcast_to(alive_scores[..., None], (B, BEAM, 128))
    out_shapes = [
        jax.ShapeDtypeStruct((B, 1, K), jnp.float32),
        jax.ShapeDtypeStruct((B, 1, K), jnp.int32),
        jax.ShapeDtypeStruct((B, 1, K), jnp.int32),
    ]
    out_spec = pl.BlockSpec((BP, 1, K), lambda b: (b, 0, 0))
    topv, beam_idx, tok_idx = pl.pallas_call(
        _beam_step_kernel,
        grid=(B // BP,),
        in_specs=[
            pl.BlockSpec((NQ, RS, 128), lambda b: (b, 0, 0)),
            pl.BlockSpec((BP, BEAM, 128), lambda b: (b, 0, 0)),
        ],
        out_specs=[out_spec, out_spec, out_spec],
        out_shape=out_shapes,
        scratch_shapes=[pltpu.VMEM((NQ, RS, 128), jnp.float32)],
    )(lp3, alive3)
    return (topv.reshape(B, K), beam_idx.reshape(B, K), tok_idx.reshape(B, K))


# speedup vs baseline: 5.8382x; 1.0863x over previous
import jax
import jax.numpy as jnp
from jax.experimental import pallas as pl

B = 32
BEAM = 16
V = 32768
K = 16
EOS_INDEX = 2
EOS_THRESHOLD = 1.5
NEG_BIG = -1e20
NEG_SENT = -3e38
BIG_I = 2 ** 30

BP = 4
SEG = 4
RS = 64
NB = BP * BEAM


def _beam_step_kernel(lp_ref, alive_ref, vals_ref, beam_ref, tok_ref):
    x = lp_ref[...]
    alive = jnp.max(alive_ref[...].reshape(NB, 128), axis=1,
                    keepdims=True)[:, :, None]

    sio4 = jax.lax.broadcasted_iota(jnp.int32, (NB, SEG, RS, 128), 1)
    rio4 = jax.lax.broadcasted_iota(jnp.int32, (NB, SEG, RS, 128), 2)
    lio4 = jax.lax.broadcasted_iota(jnp.int32, (NB, SEG, RS, 128), 3)
    pm = (sio4 == 0) & (rio4 == 0) & (lio4 == EOS_INDEX)

    c0x = jnp.max(jnp.where(pm, jnp.float32(NEG_SENT), x), axis=2)
    m0 = jnp.max(c0x, axis=(1, 2), keepdims=True)[:, :, 0, None]
    lane3 = jax.lax.broadcasted_iota(jnp.int32, (NB, 1, 128), 2)
    eosrow = x[:, 0, 0:1, :]
    eos_x = jnp.max(jnp.where(lane3 == EOS_INDEX, eosrow,
                              jnp.float32(NEG_SENT)),
                    axis=2, keepdims=True)
    m = jnp.maximum(m0, eos_x)

    se = jnp.sum(jnp.exp(x - m[:, :, :, None]), axis=(1, 2, 3),
                 keepdims=True)
    lse = m + jnp.log(se[:, :, :, 0])
    shift = alive - lse

    cond = (eos_x - lse) > (jnp.float32(EOS_THRESHOLD) * (m - lse))
    eos_new = jnp.where(cond, eos_x, jnp.float32(NEG_BIG) - shift)
    lp_ref[:, 0, 0:1, :] = jnp.where(lane3 == EOS_INDEX,
                                     jnp.broadcast_to(eos_new, (NB, 1, 128)),
                                     eosrow)

    sio2 = jax.lax.broadcasted_iota(jnp.int32, (NB, SEG, 128), 1)
    lio2 = jax.lax.broadcasted_iota(jnp.int32, (NB, SEG, 128), 2)
    c_raw = jnp.where((sio2 == 0) & (lio2 == EOS_INDEX),
                      jnp.maximum(c0x, eos_new), c0x)

    x2 = lp_ref[...]
    bio4 = jax.lax.broadcasted_iota(jnp.int32, (NB, SEG, RS, 128), 0) \
        & (BEAM - 1)
    flat4 = bio4 * V + sio4 * (RS * 128) + rio4 * 128 + lio4
    f_tab = jnp.min(jnp.where(x2 == c_raw[:, :, None, :], flat4, BIG_I),
                    axis=2)
    c_tab = c_raw + shift

    cs = [c_tab[b * BEAM:(b + 1) * BEAM] for b in range(BP)]
    fs = [f_tab[b * BEAM:(b + 1) * BEAM] for b in range(BP)]

    oiota = jax.lax.broadcasted_iota(jnp.int32, (1, K), 1)
    i16 = jax.lax.broadcasted_iota(jnp.int32, (BEAM, 1, 1), 0)
    is4 = jax.lax.broadcasted_iota(jnp.int32, (1, SEG, 1), 1)
    rowsNB = jax.lax.broadcasted_iota(jnp.int32, (NB, 1, 1), 0)
    l1 = jax.lax.broadcasted_iota(jnp.int32, (1, 128), 1)
    r2 = jax.lax.broadcasted_iota(jnp.int32, (1, RS, 128), 1)
    l2 = jax.lax.broadcasted_iota(jnp.int32, (1, RS, 128), 2)
    accs = [[jnp.zeros((1, K), jnp.float32), jnp.zeros((1, K), jnp.int32),
             jnp.zeros((1, K), jnp.int32)] for _ in range(BP)]
    for i in range(K):
        for b in range(BP):
            g = jnp.max(cs[b])
            fi = jnp.min(jnp.where(cs[b] == g, fs[b], BIG_I))
            bi = fi >> 15
            v = fi & (V - 1)
            s = v >> 13
            r = (v >> 7) & (RS - 1)
            lw = v & 127
            bq = b * BEAM + bi
            accs[b][0] = jnp.where(oiota == i, g, accs[b][0])
            accs[b][1] = jnp.where(oiota == i, bi, accs[b][1])
            accs[b][2] = jnp.where(oiota == i, v, accs[b][2])
            row = lp_ref[bq, s, pl.ds(r, 1), :]
            lp_ref[bq, s, pl.ds(r, 1), :] = jnp.where(
                l1 == lw, jnp.float32(NEG_SENT), row)
            xq = lp_ref[bq, pl.ds(s, 1)]
            crow = jnp.max(xq, axis=1)
            frow = jnp.min(jnp.where(xq == crow[:, None, :],
                                     bi * V + s * (RS * 128) + r2 * 128 + l2,
                                     BIG_I), axis=1)
            shift_b = jnp.max(jnp.where(rowsNB == bq, shift,
                                        jnp.float32(NEG_SENT)))
            sel = (i16 == bi) & (is4 == s)
            cs[b] = jnp.where(sel, (crow + shift_b)[:, None, :], cs[b])
            fs[b] = jnp.where(sel, frow[:, None, :], fs[b])

    for b in range(BP):
        vals_ref[b] = accs[b][0]
        beam_ref[b] = accs[b][1]
        tok_ref[b] = accs[b][2]


def kernel(log_probs, alive_scores, k):
    del k
    lp4 = log_probs.reshape(B * BEAM, SEG, RS, 128)
    alive3 = jnp.broadcast_to(alive_scores[..., None], (B, BEAM, 128))
    out_shapes = [
        jax.ShapeDtypeStruct((B, 1, K), jnp.float32),
        jax.ShapeDtypeStruct((B, 1, K), jnp.int32),
        jax.ShapeDtypeStruct((B, 1, K), jnp.int32),
    ]
    out_spec = pl.BlockSpec((BP, 1, K), lambda b: (b, 0, 0))
    topv, beam_idx, tok_idx = pl.pallas_call(
        _beam_step_kernel,
        grid=(B // BP,),
        in_specs=[
            pl.BlockSpec((NB, SEG, RS, 128), lambda b: (b, 0, 0, 0)),
            pl.BlockSpec((BP, BEAM, 128), lambda b: (b, 0, 0)),
        ],
        out_specs=[out_spec, out_spec, out_spec],
        out_shape=out_shapes,
    )(lp4, alive3)
    return (topv.reshape(B, K), beam_idx.reshape(B, K), tok_idx.reshape(B, K))
